# baseline (speedup 1.0000x reference)
import jax
import jax.numpy as jnp
from jax import lax
from jax.experimental import pallas as pl
from jax.experimental.pallas import tpu as pltpu
from jax.experimental.pallas import tpu_sc as plsc

NC = 2
NS = 16
NW = NC * NS
WIN = 64
NBUF = 4
G = 2
WPR = 128 // (WIN // 2)
NWIN_ALIGN = 8
WB = 80

_f32 = jnp.float32


def _sc_scatter_pass(n, d, nwin, gather):
    nacc = -(-(n + 1) // WIN) * WIN
    nzc = nacc // WIN
    zc_iters = -(-nzc // NS)
    nwb = n // WB
    assert nwb * WB == n
    wb_iters = -(-nwb // NS)
    T = nwin // NBUF
    assert T * NBUF == nwin and nwin % WPR == 0 and nwin % 8 == 0
    srows = nwin // WPR

    mesh = plsc.VectorSubcoreMesh(core_axis_name="c", subcore_axis_name="s")

    def body(*refs):
        if gather:
            (y_hbm, srcp_hbm, dstp_hbm, out_hbm, srcp, dstp, sstage, dstage,
             rows, acc_sh) = refs[:10]
            gsem = refs[10:10 + NBUF]
            ssem = refs[10 + NBUF:]
        else:
            (dstp_hbm, out_hbm, dstp, dstage, rows, acc_sh) = refs[:6]
            gsem = None
            ssem = refs[6:]
        cid = lax.axis_index("c")
        sid = lax.axis_index("s")
        wid = sid * NC + cid

        def row0():
            return rows.at[0] if gather else rows

        zero16 = jnp.zeros((16,), _f32)

        @pl.loop(0, WIN)
        def _(i):
            for j0 in range(0, d, 16):
                if gather:
                    rows[0, i, pl.ds(j0, 16)] = zero16
                else:
                    rows[i, pl.ds(j0, 16)] = zero16

        @pl.loop(0, zc_iters)
        def _(t):
            c = sid + NS * t

            @pl.when(c < nzc)
            def _():
                roff = pl.multiple_of(c * WIN, 8)
                pltpu.sync_copy(row0(), acc_sh.at[pl.ds(roff, WIN)])

        if not gather:
            one16 = jnp.full((16,), 1.0, _f32)

            @pl.loop(0, WIN)
            def _(i):
                for j0 in range(0, d, 16):
                    rows[i, pl.ds(j0, 16)] = one16

        woff = pl.multiple_of(wid * srows, 8)
        pltpu.sync_copy(dstp_hbm.at[pl.ds(woff, srows)], dstp)
        if gather:
            pltpu.sync_copy(srcp_hbm.at[pl.ds(woff, srows)], srcp)

        plsc.subcore_barrier()

        def unpack(w, slab, stage, b):
            r = w // WPR
            cb = (w % WPR) * (WIN // 2)
            for g in range(WIN // 32):
                v = slab.at[r][pl.ds(cb + 16 * g, 16)]
                stage.at[b][pl.ds(32 * g, 16)] = v & 0xFFFF
                stage.at[b][pl.ds(32 * g + 16, 16)] = (
                    lax.shift_right_logical(v, 16))

        def start_gather(b):
            pltpu.async_copy(y_hbm.at[sstage.at[b]], rows.at[b], gsem[b])

        def wait_gather(b):
            pltpu.make_async_copy(
                y_hbm.at[sstage.at[b]], rows.at[b], gsem[b]).wait()

        def start_scatter(b):
            pltpu.async_copy(row0() if not gather else rows.at[b],
                             acc_sh.at[dstage.at[b]], ssem[b], add=True)

        def wait_scatter(b):
            pltpu.make_async_copy(
                row0() if not gather else rows.at[b],
                acc_sh.at[dstage.at[b]], ssem[b]).wait()

        def stage_window(w, b):
            unpack(w, dstp, dstage, b)
            if gather:
                unpack(w, srcp, sstage, b)
                start_gather(b)

        for w0 in range(G):
            stage_window(w0, w0)

        @pl.loop(0, T - 1)
        def _(t):
            for j in range(NBUF):
                w = t * NBUF + j
                b2 = (j + G) % NBUF
                if j < G:
                    @pl.when(t > 0)
                    def _():
                        wait_scatter(b2)
                else:
                    wait_scatter(b2)
                stage_window(w + G, b2)
                if gather:
                    wait_gather(j)
                start_scatter(j)

        for j in range(NBUF):
            b2 = (j + G) % NBUF
            wait_scatter(b2)
            if j < G:
                stage_window((T - 1) * NBUF + j + G, b2)
            if gather:
                wait_gather(j)
            start_scatter(j)
        for j in range(G):
            wait_scatter((j + G) % NBUF)

        plsc.subcore_barrier()

        @pl.loop(0, wb_iters)
        def _(t):
            c = sid + NS * t

            @pl.when(c < nwb)
            def _():
                roff = pl.multiple_of(c * WB, 8)
                pltpu.sync_copy(acc_sh.at[pl.ds(roff, WB)],
                                out_hbm.at[cid, pl.ds(roff, WB)])

    scratch = []
    if gather:
        scratch += [
            pltpu.VMEM((srows, 128), jnp.int32),
            pltpu.VMEM((srows, 128), jnp.int32),
            pltpu.VMEM((NBUF, WIN), jnp.int32),
            pltpu.VMEM((NBUF, WIN), jnp.int32),
            pltpu.VMEM((NBUF, WIN, d), _f32),
        ]
    else:
        scratch += [
            pltpu.VMEM((srows, 128), jnp.int32),
            pltpu.VMEM((NBUF, WIN), jnp.int32),
            pltpu.VMEM((WIN, d), _f32),
        ]
    scratch.append(pltpu.VMEM_SHARED((nacc, d), _f32))
    nsem = 2 * NBUF if gather else NBUF
    scratch += [pltpu.SemaphoreType.DMA] * nsem
    return pl.kernel(
        body,
        out_type=jax.ShapeDtypeStruct((NC, n, d), _f32),
        mesh=mesh,
        scratch_types=scratch,
    )


def _tc_matmul_body(x_ref, w_ref, o_ref):
    o_ref[...] = jnp.dot(x_ref[...], w_ref[...],
                         preferred_element_type=_f32)


def _tc_scale_body(degp_ref, h_ref, y_ref, dinv_ref):
    deg = degp_ref[0, :, 0:1] + degp_ref[1, :, 0:1] + 1.0
    dinv = lax.rsqrt(deg)
    dinv_ref[...] = dinv
    y_ref[...] = h_ref[...] * dinv


def _tc_combine_body(p_ref, y_ref, dinv_ref, b_ref, w_ref, o_ref):
    agg = (p_ref[0] + p_ref[1] + y_ref[...]) * dinv_ref[...]
    h = jnp.maximum(agg + b_ref[...], 0.0)
    o_ref[...] = jnp.dot(h, w_ref[...], preferred_element_type=_f32) * dinv_ref[...]


def _tc_combine_nomm_body(p_ref, y_ref, dinv_ref, b_ref, o_ref):
    agg = (p_ref[0] + p_ref[1] + y_ref[...]) * dinv_ref[...]
    o_ref[...] = jnp.maximum(agg + b_ref[...], 0.0) * dinv_ref[...]


def _tc_final_body(p_ref, y_ref, dinv_ref, b_ref, w_ref, o_ref):
    agg = (p_ref[0] + p_ref[1] + y_ref[...]) * dinv_ref[...]
    z = jnp.dot(agg, w_ref[...], preferred_element_type=_f32) + b_ref[...]
    m = jnp.max(z, axis=-1, keepdims=True)
    t = z - m
    o_ref[...] = t - jnp.log(jnp.sum(jnp.exp(t), axis=-1, keepdims=True))


def _pack_idx(a):
    a = a.reshape(-1, 2, 16)
    return (a[:, 0, :] | (a[:, 1, :] << 16)).reshape(-1, 128)


def kernel(x, edge_index, W1, b1, W2, b2, W3, b3):
    n, _ = x.shape
    e = edge_index.shape[1]
    nh = W1.shape[1]
    nc = W3.shape[1]
    src = edge_index[0].astype(jnp.int32)
    dst = edge_index[1].astype(jnp.int32)

    step = NW * WIN * NWIN_ALIGN
    ep = -(-e // step) * step
    if ep != e:
        src = jnp.concatenate([src, jnp.zeros((ep - e,), jnp.int32)])
        dst = jnp.concatenate([dst, jnp.full((ep - e,), n, jnp.int32)])
    srcp = _pack_idx(src)
    dstp = _pack_idx(dst)
    nwin = ep // (NW * WIN)

    sds = jax.ShapeDtypeStruct

    h1 = pl.pallas_call(_tc_matmul_body, out_shape=sds((n, nh), _f32))(x, W1)
    degp = _sc_scatter_pass(n, 16, nwin, gather=False)(dstp)

    y1, dinv = pl.pallas_call(
        _tc_scale_body,
        out_shape=(sds((n, nh), _f32), sds((n, 1), _f32)),
    )(degp, h1)

    p1 = _sc_scatter_pass(n, nh, nwin, gather=True)(y1, srcp, dstp)
    y2 = pl.pallas_call(_tc_combine_body, out_shape=sds((n, nh), _f32))(
        p1, y1, dinv, b1.reshape(1, nh), W2)

    p2 = _sc_scatter_pass(n, nh, nwin, gather=True)(y2, srcp, dstp)
    y3 = pl.pallas_call(_tc_combine_nomm_body, out_shape=sds((n, nh), _f32))(
        p2, y2, dinv, b2.reshape(1, nh))

    p3 = _sc_scatter_pass(n, nh, nwin, gather=True)(y3, srcp, dstp)
    out = pl.pallas_call(_tc_final_body, out_shape=sds((n, nc), _f32))(
        p3, y3, dinv, b3.reshape(1, nc), W3)
    return out


# speedup vs baseline: 1.0002x; 1.0002x over previous
import jax
import jax.numpy as jnp
from jax import lax
from jax.experimental import pallas as pl
from jax.experimental.pallas import tpu as pltpu
from jax.experimental.pallas import tpu_sc as plsc

NC = 2
NS = 16
NW = NC * NS
WIN = 64
NBUF = 4
G = 2
WPR = 128 // (WIN // 2)
NWIN_ALIGN = 8
WB = 80
JUNK = 512

_f32 = jnp.float32


def _sc_scatter_pass(n, d, nwin, gather):
    nacc = -(-(n + JUNK) // WIN) * WIN
    nzc = nacc // WIN
    zc_iters = -(-nzc // NS)
    nwb = n // WB
    assert nwb * WB == n
    wb_iters = -(-nwb // NS)
    T = nwin // NBUF
    assert T * NBUF == nwin and nwin % WPR == 0 and nwin % 8 == 0
    srows = nwin // WPR

    mesh = plsc.VectorSubcoreMesh(core_axis_name="c", subcore_axis_name="s")

    def body(*refs):
        if gather:
            (y_hbm, srcp_hbm, dstp_hbm, out_hbm, srcp, dstp, sstage, dstage,
             rows, acc_sh) = refs[:10]
            gsem = refs[10:10 + NBUF]
            ssem = refs[10 + NBUF:]
        else:
            (dstp_hbm, out_hbm, dstp, dstage, rows, acc_sh) = refs[:6]
            gsem = None
            ssem = refs[6:]
        cid = lax.axis_index("c")
        sid = lax.axis_index("s")
        wid = sid * NC + cid

        def row0():
            return rows.at[0] if gather else rows

        zero16 = jnp.zeros((16,), _f32)

        @pl.loop(0, WIN)
        def _(i):
            for j0 in range(0, d, 16):
                if gather:
                    rows[0, i, pl.ds(j0, 16)] = zero16
                else:
                    rows[i, pl.ds(j0, 16)] = zero16

        @pl.loop(0, zc_iters)
        def _(t):
            c = sid + NS * t

            @pl.when(c < nzc)
            def _():
                roff = pl.multiple_of(c * WIN, 8)
                pltpu.sync_copy(row0(), acc_sh.at[pl.ds(roff, WIN)])

        if not gather:
            one16 = jnp.full((16,), 1.0, _f32)

            @pl.loop(0, WIN)
            def _(i):
                for j0 in range(0, d, 16):
                    rows[i, pl.ds(j0, 16)] = one16

        woff = pl.multiple_of(wid * srows, 8)
        pltpu.sync_copy(dstp_hbm.at[pl.ds(woff, srows)], dstp)
        if gather:
            pltpu.sync_copy(srcp_hbm.at[pl.ds(woff, srows)], srcp)

        plsc.subcore_barrier()

        def unpack(w, slab, stage, b):
            r = w // WPR
            cb = (w % WPR) * (WIN // 2)
            for g in range(WIN // 32):
                v = slab.at[r][pl.ds(cb + 16 * g, 16)]
                stage.at[b][pl.ds(32 * g, 16)] = v & 0xFFFF
                stage.at[b][pl.ds(32 * g + 16, 16)] = (
                    lax.shift_right_logical(v, 16))

        def start_gather(b):
            pltpu.async_copy(y_hbm.at[sstage.at[b]], rows.at[b], gsem[b])

        def wait_gather(b):
            pltpu.make_async_copy(
                y_hbm.at[sstage.at[b]], rows.at[b], gsem[b]).wait()

        def start_scatter(b):
            pltpu.async_copy(row0() if not gather else rows.at[b],
                             acc_sh.at[dstage.at[b]], ssem[b], add=True)

        def wait_scatter(b):
            pltpu.make_async_copy(
                row0() if not gather else rows.at[b],
                acc_sh.at[dstage.at[b]], ssem[b]).wait()

        def stage_window(w, b):
            unpack(w, dstp, dstage, b)
            if gather:
                unpack(w, srcp, sstage, b)
                start_gather(b)

        for w0 in range(G):
            stage_window(w0, w0)

        @pl.loop(0, T - 1)
        def _(t):
            for j in range(NBUF):
                w = t * NBUF + j
                b2 = (j + G) % NBUF
                if j < G:
                    @pl.when(t > 0)
                    def _():
                        wait_scatter(b2)
                else:
                    wait_scatter(b2)
                stage_window(w + G, b2)
                if gather:
                    wait_gather(j)
                start_scatter(j)

        for j in range(NBUF):
            b2 = (j + G) % NBUF
            wait_scatter(b2)
            if j < G:
                stage_window((T - 1) * NBUF + j + G, b2)
            if gather:
                wait_gather(j)
            start_scatter(j)
        for j in range(G):
            wait_scatter((j + G) % NBUF)

        plsc.subcore_barrier()

        @pl.loop(0, wb_iters)
        def _(t):
            c = sid + NS * t

            @pl.when(c < nwb)
            def _():
                roff = pl.multiple_of(c * WB, 8)
                pltpu.sync_copy(acc_sh.at[pl.ds(roff, WB)],
                                out_hbm.at[cid, pl.ds(roff, WB)])

    scratch = []
    if gather:
        scratch += [
            pltpu.VMEM((srows, 128), jnp.int32),
            pltpu.VMEM((srows, 128), jnp.int32),
            pltpu.VMEM((NBUF, WIN), jnp.int32),
            pltpu.VMEM((NBUF, WIN), jnp.int32),
            pltpu.VMEM((NBUF, WIN, d), _f32),
        ]
    else:
        scratch += [
            pltpu.VMEM((srows, 128), jnp.int32),
            pltpu.VMEM((NBUF, WIN), jnp.int32),
            pltpu.VMEM((WIN, d), _f32),
        ]
    scratch.append(pltpu.VMEM_SHARED((nacc, d), _f32))
    nsem = 2 * NBUF if gather else NBUF
    scratch += [pltpu.SemaphoreType.DMA] * nsem
    return pl.kernel(
        body,
        out_type=jax.ShapeDtypeStruct((NC, n, d), _f32),
        mesh=mesh,
        scratch_types=scratch,
    )


def _tc_matmul_body(x_ref, w_ref, o_ref):
    o_ref[...] = jnp.dot(x_ref[...], w_ref[...],
                         preferred_element_type=_f32)


def _tc_scale_body(degp_ref, h_ref, y_ref, dinv_ref):
    deg = degp_ref[0, :, 0:1] + degp_ref[1, :, 0:1] + 1.0
    dinv = lax.rsqrt(deg)
    dinv_ref[...] = dinv
    y_ref[...] = h_ref[...] * dinv


def _tc_combine_body(p_ref, y_ref, dinv_ref, b_ref, w_ref, o_ref):
    agg = (p_ref[0] + p_ref[1] + y_ref[...]) * dinv_ref[...]
    h = jnp.maximum(agg + b_ref[...], 0.0)
    o_ref[...] = jnp.dot(h, w_ref[...], preferred_element_type=_f32) * dinv_ref[...]


def _tc_combine_nomm_body(p_ref, y_ref, dinv_ref, b_ref, o_ref):
    agg = (p_ref[0] + p_ref[1] + y_ref[...]) * dinv_ref[...]
    o_ref[...] = jnp.maximum(agg + b_ref[...], 0.0) * dinv_ref[...]


def _tc_final_body(p_ref, y_ref, dinv_ref, b_ref, w_ref, o_ref):
    agg = (p_ref[0] + p_ref[1] + y_ref[...]) * dinv_ref[...]
    z = jnp.dot(agg, w_ref[...], preferred_element_type=_f32) + b_ref[...]
    m = jnp.max(z, axis=-1, keepdims=True)
    t = z - m
    o_ref[...] = t - jnp.log(jnp.sum(jnp.exp(t), axis=-1, keepdims=True))


def _pack_idx(a):
    a = a.reshape(-1, 2, 16)
    return (a[:, 0, :] | (a[:, 1, :] << 16)).reshape(-1, 128)


def kernel(x, edge_index, W1, b1, W2, b2, W3, b3):
    n, _ = x.shape
    e = edge_index.shape[1]
    nh = W1.shape[1]
    nc = W3.shape[1]
    src = edge_index[0].astype(jnp.int32)
    dst = edge_index[1].astype(jnp.int32)

    step = NW * WIN * NWIN_ALIGN
    ep = -(-e // step) * step
    if ep != e:
        src = jnp.concatenate([src, jnp.zeros((ep - e,), jnp.int32)])
        junk = n + jnp.arange(ep - e, dtype=jnp.int32) % JUNK
        dst = jnp.concatenate([dst, junk])
    srcp = _pack_idx(src)
    dstp = _pack_idx(dst)
    nwin = ep // (NW * WIN)

    sds = jax.ShapeDtypeStruct

    h1 = pl.pallas_call(_tc_matmul_body, out_shape=sds((n, nh), _f32))(x, W1)
    degp = _sc_scatter_pass(n, 16, nwin, gather=False)(dstp)

    y1, dinv = pl.pallas_call(
        _tc_scale_body,
        out_shape=(sds((n, nh), _f32), sds((n, 1), _f32)),
    )(degp, h1)

    p1 = _sc_scatter_pass(n, nh, nwin, gather=True)(y1, srcp, dstp)
    y2 = pl.pallas_call(_tc_combine_body, out_shape=sds((n, nh), _f32))(
        p1, y1, dinv, b1.reshape(1, nh), W2)

    p2 = _sc_scatter_pass(n, nh, nwin, gather=True)(y2, srcp, dstp)
    y3 = pl.pallas_call(_tc_combine_nomm_body, out_shape=sds((n, nh), _f32))(
        p2, y2, dinv, b2.reshape(1, nh))

    p3 = _sc_scatter_pass(n, nh, nwin, gather=True)(y3, srcp, dstp)
    out = pl.pallas_call(_tc_final_body, out_shape=sds((n, nc), _f32))(
        p3, y3, dinv, b3.reshape(1, nc), W3)
    return out


# speedup vs baseline: 2.8538x; 2.8533x over previous
import jax
import jax.numpy as jnp
from jax import lax
from jax.experimental import pallas as pl
from jax.experimental.pallas import tpu as pltpu
from jax.experimental.pallas import tpu_sc as plsc

NC = 2
NS = 16
NW = NC * NS
WIN = 64
NBUF = 4
G = 2
WPR = 128 // (WIN // 2)
NWIN_ALIGN = 8
WB = 80
JUNK = 512

_f32 = jnp.float32


def _sc_scatter_pass(n, d, nwin, gather):
    nacc = -(-(n + JUNK) // WIN) * WIN
    nzc = nacc // WIN
    zc_iters = -(-nzc // NS)
    nwb = n // WB
    assert nwb * WB == n
    wb_iters = -(-nwb // NS)
    T = nwin // NBUF
    assert T * NBUF == nwin and nwin % WPR == 0 and nwin % 8 == 0
    srows = nwin // WPR

    mesh = plsc.VectorSubcoreMesh(core_axis_name="c", subcore_axis_name="s")

    def body(*refs):
        if gather:
            (y_hbm, srcp_hbm, dstp_hbm, out_hbm, srcp, dstp, sstage, dstage,
             rows, acc_sh) = refs[:10]
            gsem = refs[10:10 + NBUF]
            ssem = refs[10 + NBUF:]
        else:
            (dstp_hbm, out_hbm, dstp, dstage, rows, acc_sh) = refs[:6]
            gsem = None
            ssem = refs[6:]
        cid = lax.axis_index("c")
        sid = lax.axis_index("s")
        wid = sid * NC + cid

        def row0():
            return rows.at[0] if gather else rows

        zero16 = jnp.zeros((16,), _f32)

        @pl.loop(0, WIN)
        def _(i):
            for j0 in range(0, d, 16):
                if gather:
                    rows[0, i, pl.ds(j0, 16)] = zero16
                else:
                    rows[i, pl.ds(j0, 16)] = zero16

        @pl.loop(0, zc_iters)
        def _(t):
            c = sid + NS * t

            @pl.when(c < nzc)
            def _():
                roff = pl.multiple_of(c * WIN, 8)
                pltpu.sync_copy(row0(), acc_sh.at[pl.ds(roff, WIN)])

        if not gather:
            one16 = jnp.full((16,), 1.0, _f32)

            @pl.loop(0, WIN)
            def _(i):
                for j0 in range(0, d, 16):
                    rows[i, pl.ds(j0, 16)] = one16

        woff = pl.multiple_of(wid * srows, 8)
        pltpu.sync_copy(dstp_hbm.at[pl.ds(woff, srows)], dstp)
        if gather:
            pltpu.sync_copy(srcp_hbm.at[pl.ds(woff, srows)], srcp)

        plsc.subcore_barrier()

        def unpack(w, slab, stage, b):
            r = w // WPR
            cb = (w % WPR) * (WIN // 2)
            for g in range(WIN // 32):
                v = slab.at[r][pl.ds(cb + 16 * g, 16)]
                stage.at[b][pl.ds(32 * g, 16)] = v & 0xFFFF
                stage.at[b][pl.ds(32 * g + 16, 16)] = (
                    lax.shift_right_logical(v, 16))

        def start_gather(b):
            pltpu.async_copy(y_hbm.at[sstage.at[b]], rows.at[b], gsem[b])

        def wait_gather(b):
            pltpu.make_async_copy(
                y_hbm.at[sstage.at[b]], rows.at[b], gsem[b]).wait()

        def start_scatter(b):
            pltpu.async_copy(row0() if not gather else rows.at[b],
                             acc_sh.at[dstage.at[b]], ssem[b], add=True)

        def wait_scatter(b):
            pltpu.make_async_copy(
                row0() if not gather else rows.at[b],
                acc_sh.at[dstage.at[b]], ssem[b]).wait()

        def stage_window(w, b):
            unpack(w, dstp, dstage, b)
            if gather:
                unpack(w, srcp, sstage, b)
                start_gather(b)

        for w0 in range(G):
            stage_window(w0, w0)

        @pl.loop(0, T - 1)
        def _(t):
            for j in range(NBUF):
                w = t * NBUF + j
                b2 = (j + G) % NBUF
                if j < G:
                    @pl.when(t > 0)
                    def _():
                        wait_scatter(b2)
                else:
                    wait_scatter(b2)
                stage_window(w + G, b2)
                if gather:
                    wait_gather(j)
                start_scatter(j)

        for j in range(NBUF):
            b2 = (j + G) % NBUF
            wait_scatter(b2)
            if j < G:
                stage_window((T - 1) * NBUF + j + G, b2)
            if gather:
                wait_gather(j)
            start_scatter(j)
        for j in range(G):
            wait_scatter((j + G) % NBUF)

        plsc.subcore_barrier()

        @pl.loop(0, wb_iters)
        def _(t):
            c = sid + NS * t

            @pl.when(c < nwb)
            def _():
                roff = pl.multiple_of(c * WB, 8)
                pltpu.sync_copy(acc_sh.at[pl.ds(roff, WB)],
                                out_hbm.at[cid, pl.ds(roff, WB)])

    scratch = []
    if gather:
        scratch += [
            pltpu.VMEM((srows, 128), jnp.int32),
            pltpu.VMEM((srows, 128), jnp.int32),
            pltpu.VMEM((NBUF, WIN), jnp.int32),
            pltpu.VMEM((NBUF, WIN), jnp.int32),
            pltpu.VMEM((NBUF, WIN, d), _f32),
        ]
    else:
        scratch += [
            pltpu.VMEM((srows, 128), jnp.int32),
            pltpu.VMEM((NBUF, WIN), jnp.int32),
            pltpu.VMEM((WIN, d), _f32),
        ]
    scratch.append(pltpu.VMEM_SHARED((nacc, d), _f32))
    nsem = 2 * NBUF if gather else NBUF
    scratch += [pltpu.SemaphoreType.DMA] * nsem
    return pl.kernel(
        body,
        out_type=jax.ShapeDtypeStruct((NC, n, d), _f32),
        mesh=mesh,
        scratch_types=scratch,
    )


def _tc_matmul_body(x_ref, w_ref, o_ref):
    o_ref[...] = jnp.dot(x_ref[...], w_ref[...],
                         preferred_element_type=_f32)


def _tc_scale_body(degp_ref, h_ref, y_ref, dinv_ref):
    deg = degp_ref[0, :, 0:1] + degp_ref[1, :, 0:1] + 1.0
    dinv = lax.rsqrt(deg)
    dinv_ref[...] = dinv
    y_ref[...] = h_ref[...] * dinv


def _tc_combine_body(p_ref, y_ref, dinv_ref, b_ref, w_ref, o_ref):
    agg = (p_ref[0] + p_ref[1] + y_ref[...]) * dinv_ref[...]
    h = jnp.maximum(agg + b_ref[...], 0.0)
    o_ref[...] = jnp.dot(h, w_ref[...], preferred_element_type=_f32) * dinv_ref[...]


def _tc_combine_nomm_body(p_ref, y_ref, dinv_ref, b_ref, o_ref):
    agg = (p_ref[0] + p_ref[1] + y_ref[...]) * dinv_ref[...]
    o_ref[...] = jnp.maximum(agg + b_ref[...], 0.0) * dinv_ref[...]


def _tc_final_body(p_ref, y_ref, dinv_ref, b_ref, w_ref, o_ref):
    agg = (p_ref[0] + p_ref[1] + y_ref[...]) * dinv_ref[...]
    z = jnp.dot(agg, w_ref[...], preferred_element_type=_f32) + b_ref[...]
    m = jnp.max(z, axis=-1, keepdims=True)
    t = z - m
    o_ref[...] = t - jnp.log(jnp.sum(jnp.exp(t), axis=-1, keepdims=True))


def _pack_idx(a):
    a = a.reshape(-1, 2, 16)
    return (a[:, 0, :] | (a[:, 1, :] << 16)).reshape(-1, 128)


def kernel(x, edge_index, W1, b1, W2, b2, W3, b3):
    n, _ = x.shape
    e = edge_index.shape[1]
    nh = W1.shape[1]
    nc = W3.shape[1]
    src = edge_index[0].astype(jnp.int32)
    dst = edge_index[1].astype(jnp.int32)

    step = NW * WIN * NWIN_ALIGN
    ep = -(-e // step) * step
    if ep != e:
        pad_i = jnp.arange(ep - e, dtype=jnp.int32)
        src = jnp.concatenate([src, (pad_i * 131) % n])
        dst = jnp.concatenate([dst, n + pad_i % JUNK])
    srcp = _pack_idx(src)
    dstp = _pack_idx(dst)
    nwin = ep // (NW * WIN)

    sds = jax.ShapeDtypeStruct

    h1 = pl.pallas_call(_tc_matmul_body, out_shape=sds((n, nh), _f32))(x, W1)
    degp = _sc_scatter_pass(n, 16, nwin, gather=False)(dstp)

    y1, dinv = pl.pallas_call(
        _tc_scale_body,
        out_shape=(sds((n, nh), _f32), sds((n, 1), _f32)),
    )(degp, h1)

    p1 = _sc_scatter_pass(n, nh, nwin, gather=True)(y1, srcp, dstp)
    y2 = pl.pallas_call(_tc_combine_body, out_shape=sds((n, nh), _f32))(
        p1, y1, dinv, b1.reshape(1, nh), W2)

    p2 = _sc_scatter_pass(n, nh, nwin, gather=True)(y2, srcp, dstp)
    y3 = pl.pallas_call(_tc_combine_nomm_body, out_shape=sds((n, nh), _f32))(
        p2, y2, dinv, b2.reshape(1, nh))

    p3 = _sc_scatter_pass(n, nh, nwin, gather=True)(y3, srcp, dstp)
    out = pl.pallas_call(_tc_final_body, out_shape=sds((n, nc), _f32))(
        p3, y3, dinv, b3.reshape(1, nc), W3)
    return out


# speedup vs baseline: 3.5153x; 1.2318x over previous
import jax
import jax.numpy as jnp
from jax import lax
from jax.experimental import pallas as pl
from jax.experimental.pallas import tpu as pltpu
from jax.experimental.pallas import tpu_sc as plsc

NC = 2
NS = 16
NW = NC * NS
WIN = 64
NBUF = 4
NST = 8
G = 2
IL = 4
UNROLL = 8
WB = 80
JUNK = 512

_f32 = jnp.float32


def _sc_scatter_pass(n, d, nwin, gather):
    nacc = -(-(n + JUNK) // WIN) * WIN
    nzc = nacc // WIN
    zc_iters = -(-nzc // NS)
    nwb = n // WB
    assert nwb * WB == n
    wb_iters = -(-nwb // NS)
    T = nwin // UNROLL
    assert T * UNROLL == nwin and T >= 2

    mesh = plsc.VectorSubcoreMesh(core_axis_name="c", subcore_axis_name="s")

    def body(*refs):
        if gather:
            y_hbm, src_hbm, dst_hbm, out_hbm = refs[:4]
            k = 4
            sstage = refs[k:k + NST]; k += NST
            dstage = refs[k:k + NST]; k += NST
            rows, acc_sh = refs[k:k + 2]; k += 2
            isems = refs[k:k + NBUF]; k += NBUF
            isemd = refs[k:k + NBUF]; k += NBUF
            gsem = refs[k:k + NBUF]; k += NBUF
            ssem = refs[k:k + NBUF]
        else:
            dst_hbm, out_hbm = refs[:2]
            k = 2
            dstage = refs[k:k + NST]; k += NST
            rows, acc_sh = refs[k:k + 2]; k += 2
            isemd = refs[k:k + NBUF]; k += NBUF
            ssem = refs[k:k + NBUF]
        cid = lax.axis_index("c")
        sid = lax.axis_index("s")
        wid = sid * NC + cid
        ebase = wid * nwin * WIN

        def row0():
            return rows.at[0] if gather else rows

        zero16 = jnp.zeros((16,), _f32)

        @pl.loop(0, WIN)
        def _(i):
            for j0 in range(0, d, 16):
                if gather:
                    rows[0, i, pl.ds(j0, 16)] = zero16
                else:
                    rows[i, pl.ds(j0, 16)] = zero16

        @pl.loop(0, zc_iters)
        def _(t):
            c = sid + NS * t

            @pl.when(c < nzc)
            def _():
                roff = pl.multiple_of(c * WIN, 8)
                pltpu.sync_copy(row0(), acc_sh.at[pl.ds(roff, WIN)])

        if not gather:
            one16 = jnp.full((16,), 1.0, _f32)

            @pl.loop(0, WIN)
            def _(i):
                for j0 in range(0, d, 16):
                    rows[i, pl.ds(j0, 16)] = one16

        plsc.subcore_barrier()

        def eoff(w):
            return pl.multiple_of(ebase + w * WIN, 8)

        def start_idx(w, s):
            pltpu.async_copy(dst_hbm.at[pl.ds(eoff(w), WIN)],
                             dstage[s], isemd[s % NBUF])
            if gather:
                pltpu.async_copy(src_hbm.at[pl.ds(eoff(w), WIN)],
                                 sstage[s], isems[s % NBUF])

        def wait_idx(w, s):
            pltpu.make_async_copy(dst_hbm.at[pl.ds(eoff(w), WIN)],
                                  dstage[s], isemd[s % NBUF]).wait()
            if gather:
                pltpu.make_async_copy(src_hbm.at[pl.ds(eoff(w), WIN)],
                                      sstage[s], isems[s % NBUF]).wait()

        def start_gather(s, b):
            pltpu.async_copy(y_hbm.at[sstage[s]], rows.at[b], gsem[b])

        def wait_gather(s, b):
            pltpu.make_async_copy(
                y_hbm.at[sstage[s]], rows.at[b], gsem[b]).wait()

        def payload(b):
            return rows.at[b] if gather else rows

        def start_scatter(s, b):
            pltpu.async_copy(payload(b), acc_sh.at[dstage[s]], ssem[b],
                             add=True)

        def wait_scatter(s, b):
            pltpu.make_async_copy(
                payload(b), acc_sh.at[dstage[s]], ssem[b]).wait()

        for w in range(IL):
            start_idx(w, w)
        for w in range(G):
            wait_idx(w, w)
            if gather:
                start_gather(w, w)

        def window(w, j, first):
            b = j % NBUF
            b2 = (j + G) % NBUF
            s = j % NST
            s2 = (j + G) % NST
            s4 = (j + IL) % NST
            if not first or j >= G:
                wait_scatter((j + NST - G) % NST, b2)
            start_idx(w + IL, s4)
            wait_idx(w + G, s2)
            if gather:
                start_gather(s2, b2)
                wait_gather(s, b)
            start_scatter(s, b)

        for j in range(UNROLL):
            window(j, j, first=True)

        @pl.loop(1, T - 1)
        def _(t):
            for j in range(UNROLL):
                window(t * UNROLL + j, j, first=False)

        w0 = (T - 1) * UNROLL
        for j in range(UNROLL):
            w = w0 + j
            b = j % NBUF
            b2 = (j + G) % NBUF
            s = j % NST
            s2 = (j + G) % NST
            s4 = (j + IL) % NST
            wait_scatter((j + NST - G) % NST, b2)
            if j + IL < UNROLL:
                start_idx(w + IL, s4)
            if j + G < UNROLL:
                wait_idx(w + G, s2)
                if gather:
                    start_gather(s2, b2)
            if gather:
                wait_gather(s, b)
            start_scatter(s, b)
        for j in range(G):
            wait_scatter((UNROLL - G + j) % NST, (UNROLL - G + j) % NBUF)

        plsc.subcore_barrier()

        @pl.loop(0, wb_iters)
        def _(t):
            c = sid + NS * t

            @pl.when(c < nwb)
            def _():
                roff = pl.multiple_of(c * WB, 8)
                pltpu.sync_copy(acc_sh.at[pl.ds(roff, WB)],
                                out_hbm.at[cid, pl.ds(roff, WB)])

    scratch = []
    if gather:
        scratch += [pltpu.VMEM((WIN,), jnp.int32)] * NST
    scratch += [pltpu.VMEM((WIN,), jnp.int32)] * NST
    if gather:
        scratch.append(pltpu.VMEM((NBUF, WIN, d), _f32))
    else:
        scratch.append(pltpu.VMEM((WIN, d), _f32))
    scratch.append(pltpu.VMEM_SHARED((nacc, d), _f32))
    nsem = 4 * NBUF if gather else 2 * NBUF
    scratch += [pltpu.SemaphoreType.DMA] * nsem
    return pl.kernel(
        body,
        out_type=jax.ShapeDtypeStruct((NC, n, d), _f32),
        mesh=mesh,
        scratch_types=scratch,
    )


def _tc_matmul_body(x_ref, w_ref, o_ref):
    o_ref[...] = jnp.dot(x_ref[...], w_ref[...],
                         preferred_element_type=_f32)


def _tc_scale_body(degp_ref, h_ref, y_ref, dinv_ref):
    deg = degp_ref[0, :, 0:1] + degp_ref[1, :, 0:1] + 1.0
    dinv = lax.rsqrt(deg)
    dinv_ref[...] = dinv
    y_ref[...] = h_ref[...] * dinv


def _tc_combine_body(p_ref, y_ref, dinv_ref, b_ref, w_ref, o_ref):
    agg = (p_ref[0] + p_ref[1] + y_ref[...]) * dinv_ref[...]
    h = jnp.maximum(agg + b_ref[...], 0.0)
    o_ref[...] = jnp.dot(h, w_ref[...], preferred_element_type=_f32) * dinv_ref[...]


def _tc_combine_nomm_body(p_ref, y_ref, dinv_ref, b_ref, o_ref):
    agg = (p_ref[0] + p_ref[1] + y_ref[...]) * dinv_ref[...]
    o_ref[...] = jnp.maximum(agg + b_ref[...], 0.0) * dinv_ref[...]


def _tc_final_body(p_ref, y_ref, dinv_ref, b_ref, w_ref, o_ref):
    agg = (p_ref[0] + p_ref[1] + y_ref[...]) * dinv_ref[...]
    z = jnp.dot(agg, w_ref[...], preferred_element_type=_f32) + b_ref[...]
    m = jnp.max(z, axis=-1, keepdims=True)
    t = z - m
    o_ref[...] = t - jnp.log(jnp.sum(jnp.exp(t), axis=-1, keepdims=True))


def kernel(x, edge_index, W1, b1, W2, b2, W3, b3):
    n, _ = x.shape
    e = edge_index.shape[1]
    nh = W1.shape[1]
    nc = W3.shape[1]
    src = edge_index[0].astype(jnp.int32)
    dst = edge_index[1].astype(jnp.int32)

    step = NW * WIN * UNROLL
    ep = -(-e // step) * step
    if ep != e:
        pad_i = jnp.arange(ep - e, dtype=jnp.int32)
        src = jnp.concatenate([src, (pad_i * 131) % n])
        dst = jnp.concatenate([dst, n + pad_i % JUNK])
    nwin = ep // (NW * WIN)

    sds = jax.ShapeDtypeStruct

    h1 = pl.pallas_call(_tc_matmul_body, out_shape=sds((n, nh), _f32))(x, W1)
    degp = _sc_scatter_pass(n, 16, nwin, gather=False)(dst)

    y1, dinv = pl.pallas_call(
        _tc_scale_body,
        out_shape=(sds((n, nh), _f32), sds((n, 1), _f32)),
    )(degp, h1)

    p1 = _sc_scatter_pass(n, nh, nwin, gather=True)(y1, src, dst)
    y2 = pl.pallas_call(_tc_combine_body, out_shape=sds((n, nh), _f32))(
        p1, y1, dinv, b1.reshape(1, nh), W2)

    p2 = _sc_scatter_pass(n, nh, nwin, gather=True)(y2, src, dst)
    y3 = pl.pallas_call(_tc_combine_nomm_body, out_shape=sds((n, nh), _f32))(
        p2, y2, dinv, b2.reshape(1, nh))

    p3 = _sc_scatter_pass(n, nh, nwin, gather=True)(y3, src, dst)
    out = pl.pallas_call(_tc_final_body, out_shape=sds((n, nc), _f32))(
        p3, y3, dinv, b3.reshape(1, nc), W3)
    return out
